# baseline (speedup 1.0000x reference)
import functools

import jax
import jax.numpy as jnp
from jax.experimental import pallas as pl
from jax.experimental.pallas import tpu as pltpu

NBUF = 8
OBUF = 4
BN = 128


def _spatial_conv_kernel(g_hbm, x_ref, w_ref, b_ref, o_hbm,
                         buf, s_ref, y_ref, ostg, isem, osem,
                         *, n, batch, tc):
    nblk = n // BN

    def fetch(idx):
        return pltpu.make_async_copy(
            g_hbm.at[pl.ds((idx % nblk) * BN, BN), :], buf.at[idx % NBUF],
            isem.at[idx % NBUF])

    def store(idx):
        return pltpu.make_async_copy(
            ostg.at[idx % OBUF], o_hbm.at[:, pl.ds(idx * BN, BN), :],
            osem.at[idx % OBUF])

    for k in range(NBUF):
        fetch(k).start()

    def p1_body(i, carry):
        fetch(i).wait()
        s_ref[pl.ds(i * BN, BN), :] = jnp.sum(buf[i % NBUF], axis=1,
                                              keepdims=True)
        fetch(i + NBUF).start()
        return carry

    jax.lax.fori_loop(0, nblk, p1_body, 0)

    d_all = jax.lax.rsqrt(s_ref[...])
    s_ref[...] = d_all
    y_ref[...] = (x_ref[...] * d_all).astype(jnp.bfloat16)

    w = w_ref[...]
    bias = b_ref[...]
    y = y_ref[...]

    def p2_body(i, carry):
        fetch(nblk + i).wait()

        @pl.when(i >= OBUF)
        def _():
            store(i - OBUF).wait()

        z = jnp.dot(buf[i % NBUF].astype(jnp.bfloat16), y,
                    preferred_element_type=jnp.float32)
        r = i * BN
        zp = z * s_ref[pl.ds(r, BN), :]
        x = x_ref[pl.ds(r, BN), :]
        slot = i % OBUF
        for b in range(batch):
            sl = slice(b * tc, (b + 1) * tc)
            sb = jnp.concatenate([x[:, sl], zp[:, sl]], axis=1)
            ob = jnp.dot(sb, w, preferred_element_type=jnp.float32) + bias
            ostg[slot, b] = jnp.maximum(ob, 0.0)
        store(i).start()

        @pl.when(nblk + i + NBUF < 2 * nblk)
        def _():
            fetch(nblk + i + NBUF).start()

        return carry

    jax.lax.fori_loop(0, nblk, p2_body, 0)

    for k in range(OBUF):
        store(nblk - OBUF + k).wait()


def kernel(inputs, graph, weight, bias):
    B, N, T, C = inputs.shape
    D = weight.shape[-1]
    BTC = B * T * C
    TC = T * C
    TD = T * D

    x2 = inputs.transpose(1, 0, 2, 3).reshape(N, BTC)

    w0 = weight[0, 0]
    w1 = weight[1, 0]
    eye = jnp.eye(T, dtype=weight.dtype)
    wa = (eye[:, None, :, None] * (w0 + w1)[None, :, None, :]).reshape(TC, TD)
    wb = (eye[:, None, :, None] * (-w1)[None, :, None, :]).reshape(TC, TD)
    wbig = jnp.concatenate([wa, wb], axis=0)
    bias_t = jnp.tile(bias.reshape(1, D), (1, T))

    out = pl.pallas_call(
        functools.partial(_spatial_conv_kernel, n=N, batch=B, tc=TC),
        in_specs=[
            pl.BlockSpec(memory_space=pltpu.HBM),
            pl.BlockSpec(memory_space=pltpu.VMEM),
            pl.BlockSpec(memory_space=pltpu.VMEM),
            pl.BlockSpec(memory_space=pltpu.VMEM),
        ],
        out_specs=pl.BlockSpec(memory_space=pltpu.HBM),
        out_shape=jax.ShapeDtypeStruct((B, N, TD), jnp.float32),
        scratch_shapes=[
            pltpu.VMEM((NBUF, BN, N), jnp.float32),
            pltpu.VMEM((N, 1), jnp.float32),
            pltpu.VMEM((N, BTC), jnp.bfloat16),
            pltpu.VMEM((OBUF, B, BN, TD), jnp.float32),
            pltpu.SemaphoreType.DMA((NBUF,)),
            pltpu.SemaphoreType.DMA((OBUF,)),
        ],
        compiler_params=pltpu.CompilerParams(
            vmem_limit_bytes=128 * 1024 * 1024,
        ),
    )(graph, x2, wbig, bias_t)

    return out.reshape(B, N, T, D)
